# baseline (speedup 1.0000x reference)
import functools

import jax
import jax.numpy as jnp
from jax import lax
from jax.experimental import pallas as pl
from jax.experimental.pallas import tpu as pltpu
from jax.experimental.pallas import tpu_sc as plsc

NC = 2
NS = 16
NW = NC * NS
B = 96


def _mesh():
    return plsc.VectorSubcoreMesh(
        core_axis_name="c", subcore_axis_name="s", num_cores=NC, num_subcores=NS
    )


def _make_deg(N, EPW):
    NB = EPW // B
    rpt = N // NS

    @functools.partial(
        pl.kernel,
        out_type=jax.ShapeDtypeStruct((NC, N, 128), jnp.float32),
        mesh=_mesh(),
        scratch_types=[
            pltpu.VMEM((NB, B), jnp.int32),
            pltpu.VMEM((B, 128), jnp.float32),
            pltpu.VMEM_SHARED((N, 128), jnp.float32),
            pltpu.SemaphoreType.DMA,
        ],
    )
    def deg_kernel(dst_hbm, zeros_hbm, ones_hbm, out_hbm, dst_v, ones_v, acc, sem):
        c = lax.axis_index("c")
        s = lax.axis_index("s")
        wid = s * NC + c
        pltpu.sync_copy(zeros_hbm.at[pl.ds(s * rpt, rpt)], acc.at[pl.ds(s * rpt, rpt)])
        pltpu.sync_copy(dst_hbm.at[wid], dst_v)
        pltpu.sync_copy(ones_hbm, ones_v)
        plsc.subcore_barrier()

        @pl.loop(0, NB)
        def _(j):
            pltpu.sync_copy(ones_v, acc.at[dst_v.at[j]], add=True)

        plsc.subcore_barrier()
        pltpu.sync_copy(
            acc.at[pl.ds(s * rpt, rpt)], out_hbm.at[c, pl.ds(s * rpt, rpt)]
        )

    return deg_kernel


def _make_agg(N, D, EPW):
    NB = EPW // B
    rpt = N // NS

    @functools.partial(
        pl.kernel,
        out_type=jax.ShapeDtypeStruct((NC, N, D), jnp.float32),
        mesh=_mesh(),
        scratch_types=[
            pltpu.VMEM((EPW,), jnp.int32),
            pltpu.VMEM((NB, B), jnp.int32),
            pltpu.VMEM((B, D), jnp.float32),
            pltpu.VMEM((B, D), jnp.float32),
            pltpu.VMEM_SHARED((N, D), jnp.float32),
            pltpu.SemaphoreType.DMA,
            pltpu.SemaphoreType.DMA,
        ],
    )
    def agg_kernel(
        g_hbm, src_hbm, dst_hbm, zeros_hbm, out_hbm,
        src_v, dst_v, buf0, buf1, acc, semg0, semg1,
    ):
        c = lax.axis_index("c")
        s = lax.axis_index("s")
        wid = s * NC + c
        pltpu.sync_copy(zeros_hbm.at[pl.ds(s * rpt, rpt)], acc.at[pl.ds(s * rpt, rpt)])
        pltpu.sync_copy(src_hbm.at[pl.ds(wid * EPW, EPW)], src_v)
        pltpu.sync_copy(dst_hbm.at[wid], dst_v)
        plsc.subcore_barrier()

        def gather(j, buf, sem):
            pltpu.async_copy(g_hbm.at[src_v.at[pl.ds(j * B, B)]], buf, sem)

        def wait_gather(buf, sem):
            pltpu.make_async_copy(g_hbm.at[src_v.at[pl.ds(0, B)]], buf, sem).wait()

        gather(0, buf0, semg0)

        @pl.loop(0, (NB - 1) // 2)
        def _(i):
            j0 = 2 * i
            j1 = 2 * i + 1
            gather(j1, buf1, semg1)
            wait_gather(buf0, semg0)
            pltpu.sync_copy(buf0, acc.at[dst_v.at[j0]], add=True)
            gather(j0 + 2, buf0, semg0)
            wait_gather(buf1, semg1)
            pltpu.sync_copy(buf1, acc.at[dst_v.at[j1]], add=True)

        wait_gather(buf0, semg0)
        pltpu.sync_copy(buf0, acc.at[dst_v.at[NB - 1]], add=True)

        plsc.subcore_barrier()
        pltpu.sync_copy(
            acc.at[pl.ds(s * rpt, rpt)], out_hbm.at[c, pl.ds(s * rpt, rpt)]
        )

    return agg_kernel


_R = 1000


def _tc_first(x, W, degp):
    N, K = x.shape
    M = W.shape[1]

    def body(x_ref, w_ref, d_ref, g_ref, dinv_ref):
        deg = d_ref[0, :, 0:1] + d_ref[1, :, 0:1] + 1.0
        dinv = lax.rsqrt(deg)
        dinv_ref[...] = dinv
        h = jnp.dot(x_ref[...], w_ref[...], preferred_element_type=jnp.float32,
                    precision=lax.Precision.HIGHEST)
        g_ref[...] = dinv * h

    return pl.pallas_call(
        body,
        grid=(N // _R,),
        in_specs=[
            pl.BlockSpec((_R, K), lambda i: (i, 0)),
            pl.BlockSpec((K, M), lambda i: (0, 0)),
            pl.BlockSpec((NC, _R, 128), lambda i: (0, i, 0)),
        ],
        out_specs=[
            pl.BlockSpec((_R, M), lambda i: (i, 0)),
            pl.BlockSpec((_R, 1), lambda i: (i, 0)),
        ],
        out_shape=[
            jax.ShapeDtypeStruct((N, M), jnp.float32),
            jax.ShapeDtypeStruct((N, 1), jnp.float32),
        ],
    )(x, W, degp)


def _tc_mid(agg, g, dinv, b, W):
    N, D = g.shape
    M = W.shape[1]

    def body(a_ref, g_ref, dinv_ref, b_ref, w_ref, o_ref):
        dinv = dinv_ref[...]
        h = dinv * (a_ref[0] + a_ref[1] + g_ref[...]) + b_ref[...]
        h = jnp.maximum(h, 0.0)
        o_ref[...] = dinv * jnp.dot(h, w_ref[...], preferred_element_type=jnp.float32,
                                    precision=lax.Precision.HIGHEST)

    return pl.pallas_call(
        body,
        grid=(N // _R,),
        in_specs=[
            pl.BlockSpec((NC, _R, D), lambda i: (0, i, 0)),
            pl.BlockSpec((_R, D), lambda i: (i, 0)),
            pl.BlockSpec((_R, 1), lambda i: (i, 0)),
            pl.BlockSpec((1, D), lambda i: (0, 0)),
            pl.BlockSpec((D, M), lambda i: (0, 0)),
        ],
        out_specs=pl.BlockSpec((_R, M), lambda i: (i, 0)),
        out_shape=jax.ShapeDtypeStruct((N, M), jnp.float32),
    )(agg, g, dinv, b.reshape(1, D), W)


def _tc_final(agg, g, dinv, b):
    N, D = g.shape
    H = D // 2

    def body(a_ref, g_ref, dinv_ref, b_ref, mu_ref, ls_ref):
        o = dinv_ref[...] * (a_ref[0] + a_ref[1] + g_ref[...]) + b_ref[...]
        mu_ref[...] = o[:, :H]
        ls_ref[...] = o[:, H:]

    return pl.pallas_call(
        body,
        grid=(N // _R,),
        in_specs=[
            pl.BlockSpec((NC, _R, D), lambda i: (0, i, 0)),
            pl.BlockSpec((_R, D), lambda i: (i, 0)),
            pl.BlockSpec((_R, 1), lambda i: (i, 0)),
            pl.BlockSpec((1, D), lambda i: (0, 0)),
        ],
        out_specs=[
            pl.BlockSpec((_R, H), lambda i: (i, 0)),
            pl.BlockSpec((_R, H), lambda i: (i, 0)),
        ],
        out_shape=[
            jax.ShapeDtypeStruct((N, H), jnp.float32),
            jax.ShapeDtypeStruct((N, H), jnp.float32),
        ],
    )(agg, g, dinv, b.reshape(1, D))


def kernel(x, edge_index, W1, b1, W2, b2, W_mu, b_mu, W_ls, b_ls):
    N, _ = x.shape
    E = edge_index.shape[1]
    NP = -(-N // (8 * NS)) * (8 * NS)
    EPW0 = E // NW
    EPW = -(-EPW0 // B) * B
    if (EPW // B) % 2 == 0:
        EPW += B

    src = jnp.pad(edge_index[0].reshape(NW, EPW0),
                  ((0, 0), (0, EPW - EPW0))).reshape(-1)
    npad = EPW - EPW0
    if npad:
        w = jnp.arange(NW)[:, None]
        i = jnp.arange(npad)[None, :]
        trash = N + (w * 37 + i) % (NP - N)
        dst = jnp.concatenate(
            [edge_index[1].reshape(NW, EPW0), trash.astype(jnp.int32)], axis=1
        ).reshape(NW, EPW // B, B)
    else:
        dst = edge_index[1].reshape(NW, EPW // B, B)
    zeros128 = jnp.zeros((NP, 128), jnp.float32)
    ones128 = jnp.ones((B, 128), jnp.float32)

    degp = _make_deg(NP, EPW)(dst, zeros128, ones128)
    g1, dinv = _tc_first(x, W1, degp)

    agg = _make_agg(NP, 128, EPW)
    a1 = agg(g1, src, dst, zeros128)
    g2 = _tc_mid(a1, g1, dinv, b1, W2)
    a2 = agg(g2, src, dst, zeros128)
    W3 = jnp.concatenate([W_mu, W_ls], axis=1)
    b3 = jnp.concatenate([b_mu, b_ls])
    g3 = _tc_mid(a2, g2, dinv, b2, W3)
    a3 = agg(g3, src, dst, zeros128)
    return _tc_final(a3, g3, dinv, b3)


# speedup vs baseline: 1.0977x; 1.0977x over previous
import functools

import jax
import jax.numpy as jnp
from jax import lax
from jax.experimental import pallas as pl
from jax.experimental.pallas import tpu as pltpu
from jax.experimental.pallas import tpu_sc as plsc

NC = 2
NS = 16
NW = NC * NS
B = 64


def _mesh():
    return plsc.VectorSubcoreMesh(
        core_axis_name="c", subcore_axis_name="s", num_cores=NC, num_subcores=NS
    )


def _make_deg(N, EPW):
    NB = EPW // B
    rpt = N // NS

    @functools.partial(
        pl.kernel,
        out_type=jax.ShapeDtypeStruct((NC, N, 128), jnp.float32),
        mesh=_mesh(),
        scratch_types=[
            pltpu.VMEM((NB, B), jnp.int32),
            pltpu.VMEM((B, 128), jnp.float32),
            pltpu.VMEM_SHARED((N, 128), jnp.float32),
            pltpu.SemaphoreType.DMA,
        ],
    )
    def deg_kernel(dst_hbm, zeros_hbm, ones_hbm, out_hbm, dst_v, ones_v, acc, sem):
        c = lax.axis_index("c")
        s = lax.axis_index("s")
        wid = s * NC + c
        pltpu.sync_copy(zeros_hbm.at[pl.ds(s * rpt, rpt)], acc.at[pl.ds(s * rpt, rpt)])
        pltpu.sync_copy(dst_hbm.at[wid], dst_v)
        pltpu.sync_copy(ones_hbm, ones_v)
        plsc.subcore_barrier()

        @pl.loop(0, NB)
        def _(j):
            pltpu.sync_copy(ones_v, acc.at[dst_v.at[j]], add=True)

        plsc.subcore_barrier()
        pltpu.sync_copy(
            acc.at[pl.ds(s * rpt, rpt)], out_hbm.at[c, pl.ds(s * rpt, rpt)]
        )

    return deg_kernel


def _make_agg(N, D, EPW):
    NB = EPW // B
    rpt = N // NS

    @functools.partial(
        pl.kernel,
        out_type=jax.ShapeDtypeStruct((NC, N, D), jnp.float32),
        mesh=_mesh(),
        scratch_types=[
            pltpu.VMEM((EPW,), jnp.int32),
            pltpu.VMEM((NB, B), jnp.int32),
            pltpu.VMEM((B, D), jnp.float32),
            pltpu.VMEM((B, D), jnp.float32),
            pltpu.VMEM_SHARED((N, D), jnp.float32),
            pltpu.SemaphoreType.DMA,
            pltpu.SemaphoreType.DMA,
        ],
    )
    def agg_kernel(
        g_hbm, src_hbm, dst_hbm, zeros_hbm, out_hbm,
        src_v, dst_v, buf0, buf1, acc, semg0, semg1,
    ):
        c = lax.axis_index("c")
        s = lax.axis_index("s")
        wid = s * NC + c
        pltpu.sync_copy(zeros_hbm.at[pl.ds(s * rpt, rpt)], acc.at[pl.ds(s * rpt, rpt)])
        pltpu.sync_copy(src_hbm.at[pl.ds(wid * EPW, EPW)], src_v)
        pltpu.sync_copy(dst_hbm.at[wid], dst_v)
        plsc.subcore_barrier()

        def gather(j, buf, sem):
            pltpu.async_copy(g_hbm.at[src_v.at[pl.ds(j * B, B)]], buf, sem)

        def wait_gather(buf, sem):
            pltpu.make_async_copy(g_hbm.at[src_v.at[pl.ds(0, B)]], buf, sem).wait()

        gather(0, buf0, semg0)

        @pl.loop(0, (NB - 1) // 2)
        def _(i):
            j0 = 2 * i
            j1 = 2 * i + 1
            gather(j1, buf1, semg1)
            wait_gather(buf0, semg0)
            pltpu.sync_copy(buf0, acc.at[dst_v.at[j0]], add=True)
            gather(j0 + 2, buf0, semg0)
            wait_gather(buf1, semg1)
            pltpu.sync_copy(buf1, acc.at[dst_v.at[j1]], add=True)

        wait_gather(buf0, semg0)
        pltpu.sync_copy(buf0, acc.at[dst_v.at[NB - 1]], add=True)

        plsc.subcore_barrier()
        pltpu.sync_copy(
            acc.at[pl.ds(s * rpt, rpt)], out_hbm.at[c, pl.ds(s * rpt, rpt)]
        )

    return agg_kernel


_R = 1000


def _tc_first(x, W, degp):
    N, K = x.shape
    M = W.shape[1]

    def body(x_ref, w_ref, d_ref, g_ref, dinv_ref):
        deg = d_ref[0, :, 0:1] + d_ref[1, :, 0:1] + 1.0
        dinv = lax.rsqrt(deg)
        dinv_ref[...] = dinv
        h = jnp.dot(x_ref[...], w_ref[...], preferred_element_type=jnp.float32,
                    precision=lax.Precision.HIGHEST)
        g_ref[...] = dinv * h

    return pl.pallas_call(
        body,
        grid=(N // _R,),
        in_specs=[
            pl.BlockSpec((_R, K), lambda i: (i, 0)),
            pl.BlockSpec((K, M), lambda i: (0, 0)),
            pl.BlockSpec((NC, _R, 128), lambda i: (0, i, 0)),
        ],
        out_specs=[
            pl.BlockSpec((_R, M), lambda i: (i, 0)),
            pl.BlockSpec((_R, 1), lambda i: (i, 0)),
        ],
        out_shape=[
            jax.ShapeDtypeStruct((N, M), jnp.float32),
            jax.ShapeDtypeStruct((N, 1), jnp.float32),
        ],
    )(x, W, degp)


def _tc_mid(agg, g, dinv, b, W):
    N, D = g.shape
    M = W.shape[1]

    def body(a_ref, g_ref, dinv_ref, b_ref, w_ref, o_ref):
        dinv = dinv_ref[...]
        h = dinv * (a_ref[0] + a_ref[1] + g_ref[...]) + b_ref[...]
        h = jnp.maximum(h, 0.0)
        o_ref[...] = dinv * jnp.dot(h, w_ref[...], preferred_element_type=jnp.float32,
                                    precision=lax.Precision.HIGHEST)

    return pl.pallas_call(
        body,
        grid=(N // _R,),
        in_specs=[
            pl.BlockSpec((NC, _R, D), lambda i: (0, i, 0)),
            pl.BlockSpec((_R, D), lambda i: (i, 0)),
            pl.BlockSpec((_R, 1), lambda i: (i, 0)),
            pl.BlockSpec((1, D), lambda i: (0, 0)),
            pl.BlockSpec((D, M), lambda i: (0, 0)),
        ],
        out_specs=pl.BlockSpec((_R, M), lambda i: (i, 0)),
        out_shape=jax.ShapeDtypeStruct((N, M), jnp.float32),
    )(agg, g, dinv, b.reshape(1, D), W)


def _tc_final(agg, g, dinv, b):
    N, D = g.shape
    H = D // 2

    def body(a_ref, g_ref, dinv_ref, b_ref, mu_ref, ls_ref):
        o = dinv_ref[...] * (a_ref[0] + a_ref[1] + g_ref[...]) + b_ref[...]
        mu_ref[...] = o[:, :H]
        ls_ref[...] = o[:, H:]

    return pl.pallas_call(
        body,
        grid=(N // _R,),
        in_specs=[
            pl.BlockSpec((NC, _R, D), lambda i: (0, i, 0)),
            pl.BlockSpec((_R, D), lambda i: (i, 0)),
            pl.BlockSpec((_R, 1), lambda i: (i, 0)),
            pl.BlockSpec((1, D), lambda i: (0, 0)),
        ],
        out_specs=[
            pl.BlockSpec((_R, H), lambda i: (i, 0)),
            pl.BlockSpec((_R, H), lambda i: (i, 0)),
        ],
        out_shape=[
            jax.ShapeDtypeStruct((N, H), jnp.float32),
            jax.ShapeDtypeStruct((N, H), jnp.float32),
        ],
    )(agg, g, dinv, b.reshape(1, D))


def kernel(x, edge_index, W1, b1, W2, b2, W_mu, b_mu, W_ls, b_ls):
    N, _ = x.shape
    E = edge_index.shape[1]
    NP = -(-N // (8 * NS)) * (8 * NS)
    EPW0 = E // NW
    EPW = -(-EPW0 // B) * B
    if (EPW // B) % 2 == 0:
        EPW += B

    src = jnp.pad(edge_index[0].reshape(NW, EPW0),
                  ((0, 0), (0, EPW - EPW0))).reshape(-1)
    npad = EPW - EPW0
    if npad:
        w = jnp.arange(NW)[:, None]
        i = jnp.arange(npad)[None, :]
        trash = N + (w * 37 + i) % (NP - N)
        dst = jnp.concatenate(
            [edge_index[1].reshape(NW, EPW0), trash.astype(jnp.int32)], axis=1
        ).reshape(NW, EPW // B, B)
    else:
        dst = edge_index[1].reshape(NW, EPW // B, B)
    zeros128 = jnp.zeros((NP, 128), jnp.float32)
    ones128 = jnp.ones((B, 128), jnp.float32)

    degp = _make_deg(NP, EPW)(dst, zeros128, ones128)
    g1, dinv = _tc_first(x, W1, degp)

    agg = _make_agg(NP, 128, EPW)
    a1 = agg(g1, src, dst, zeros128)
    g2 = _tc_mid(a1, g1, dinv, b1, W2)
    a2 = agg(g2, src, dst, zeros128)
    W3 = jnp.concatenate([W_mu, W_ls], axis=1)
    b3 = jnp.concatenate([b_mu, b_ls])
    g3 = _tc_mid(a2, g2, dinv, b2, W3)
    a3 = agg(g3, src, dst, zeros128)
    return _tc_final(a3, g3, dinv, b3)


# speedup vs baseline: 1.4953x; 1.3622x over previous
import functools

import jax
import jax.numpy as jnp
from jax import lax
from jax.experimental import pallas as pl
from jax.experimental.pallas import tpu as pltpu
from jax.experimental.pallas import tpu_sc as plsc

NC = 2
NS = 16
NW = NC * NS
B = 80


def _mesh():
    return plsc.VectorSubcoreMesh(
        core_axis_name="c", subcore_axis_name="s", num_cores=NC, num_subcores=NS
    )


def _make_deg(N, EPW):
    NB = EPW // B
    rpt = N // NS

    @functools.partial(
        pl.kernel,
        out_type=jax.ShapeDtypeStruct((NC, N, 128), jnp.float32),
        mesh=_mesh(),
        scratch_types=[
            pltpu.VMEM((NB, B), jnp.int32),
            pltpu.VMEM((B, 128), jnp.float32),
            pltpu.VMEM_SHARED((N, 128), jnp.float32),
        ],
    )
    def deg_kernel(dst_hbm, zeros_hbm, ones_hbm, out_hbm, dst_v, ones_v, acc):
        c = lax.axis_index("c")
        s = lax.axis_index("s")
        wid = s * NC + c
        pltpu.sync_copy(zeros_hbm.at[pl.ds(s * rpt, rpt)], acc.at[pl.ds(s * rpt, rpt)])
        pltpu.sync_copy(dst_hbm.at[wid], dst_v)
        pltpu.sync_copy(ones_hbm, ones_v)
        plsc.subcore_barrier()

        @pl.loop(0, NB)
        def _(j):
            pltpu.sync_copy(ones_v, acc.at[dst_v.at[j]], add=True)

        plsc.subcore_barrier()
        pltpu.sync_copy(
            acc.at[pl.ds(s * rpt, rpt)], out_hbm.at[c, pl.ds(s * rpt, rpt)]
        )

    return deg_kernel


def _make_agg(N, D, EPW):
    NB = EPW // B
    rpt = N // NS

    @functools.partial(
        pl.kernel,
        out_type=jax.ShapeDtypeStruct((NC, N, D), jnp.float32),
        mesh=_mesh(),
        scratch_types=[
            pltpu.VMEM((EPW,), jnp.int32),
            pltpu.VMEM((NB, B), jnp.int32),
            pltpu.VMEM((B, D), jnp.float32),
            pltpu.VMEM((B, D), jnp.float32),
            pltpu.VMEM_SHARED((N, D), jnp.float32),
            pltpu.SemaphoreType.DMA,
            pltpu.SemaphoreType.DMA,
        ],
    )
    def agg_kernel(
        g_hbm, src_hbm, dst_hbm, zeros_hbm, out_hbm,
        src_v, dst_v, buf0, buf1, acc, semg0, semg1,
    ):
        c = lax.axis_index("c")
        s = lax.axis_index("s")
        wid = s * NC + c
        pltpu.sync_copy(zeros_hbm.at[pl.ds(s * rpt, rpt)], acc.at[pl.ds(s * rpt, rpt)])
        pltpu.sync_copy(src_hbm.at[pl.ds(wid * EPW, EPW)], src_v)
        pltpu.sync_copy(dst_hbm.at[wid], dst_v)
        plsc.subcore_barrier()

        def gather(j, buf, sem):
            pltpu.async_copy(g_hbm.at[src_v.at[pl.ds(j * B, B)]], buf, sem)

        def wait_gather(buf, sem):
            pltpu.make_async_copy(g_hbm.at[src_v.at[pl.ds(0, B)]], buf, sem).wait()

        gather(0, buf0, semg0)

        @pl.loop(0, (NB - 1) // 2)
        def _(i):
            j0 = 2 * i
            j1 = 2 * i + 1
            gather(j1, buf1, semg1)
            wait_gather(buf0, semg0)
            pltpu.sync_copy(buf0, acc.at[dst_v.at[j0]], add=True)
            gather(j0 + 2, buf0, semg0)
            wait_gather(buf1, semg1)
            pltpu.sync_copy(buf1, acc.at[dst_v.at[j1]], add=True)

        wait_gather(buf0, semg0)
        pltpu.sync_copy(buf0, acc.at[dst_v.at[NB - 1]], add=True)

        plsc.subcore_barrier()
        pltpu.sync_copy(
            acc.at[pl.ds(s * rpt, rpt)], out_hbm.at[c, pl.ds(s * rpt, rpt)]
        )

    return agg_kernel


_R = 1000


def _tc_matmul(x, W):
    N, K = x.shape
    M = W.shape[1]

    def body(x_ref, w_ref, h_ref):
        h_ref[...] = jnp.dot(x_ref[...], w_ref[...],
                             preferred_element_type=jnp.float32,
                             precision=lax.Precision.HIGHEST)

    return pl.pallas_call(
        body,
        grid=(N // _R,),
        in_specs=[
            pl.BlockSpec((_R, K), lambda i: (i, 0)),
            pl.BlockSpec((K, M), lambda i: (0, 0)),
        ],
        out_specs=pl.BlockSpec((_R, M), lambda i: (i, 0)),
        out_shape=jax.ShapeDtypeStruct((N, M), jnp.float32),
    )(x, W)


def _tc_scale(h, degp):
    N, M = h.shape

    def body(h_ref, d_ref, g_ref, dinv_ref):
        deg = d_ref[0, :, 0:1] + d_ref[1, :, 0:1] + 1.0
        dinv = lax.rsqrt(deg)
        dinv_ref[...] = dinv
        g_ref[...] = dinv * h_ref[...]

    return pl.pallas_call(
        body,
        grid=(N // _R,),
        in_specs=[
            pl.BlockSpec((_R, M), lambda i: (i, 0)),
            pl.BlockSpec((NC, _R, 128), lambda i: (0, i, 0)),
        ],
        out_specs=[
            pl.BlockSpec((_R, M), lambda i: (i, 0)),
            pl.BlockSpec((_R, 1), lambda i: (i, 0)),
        ],
        out_shape=[
            jax.ShapeDtypeStruct((N, M), jnp.float32),
            jax.ShapeDtypeStruct((N, 1), jnp.float32),
        ],
    )(h, degp)


def _tc_mid(agg, g, dinv, b, W):
    N, D = g.shape
    M = W.shape[1]

    def body(a_ref, g_ref, dinv_ref, b_ref, w_ref, o_ref):
        dinv = dinv_ref[...]
        h = dinv * (a_ref[0] + a_ref[1] + g_ref[...]) + b_ref[...]
        h = jnp.maximum(h, 0.0)
        o_ref[...] = dinv * jnp.dot(h, w_ref[...], preferred_element_type=jnp.float32,
                                    precision=lax.Precision.HIGHEST)

    return pl.pallas_call(
        body,
        grid=(N // _R,),
        in_specs=[
            pl.BlockSpec((NC, _R, D), lambda i: (0, i, 0)),
            pl.BlockSpec((_R, D), lambda i: (i, 0)),
            pl.BlockSpec((_R, 1), lambda i: (i, 0)),
            pl.BlockSpec((1, D), lambda i: (0, 0)),
            pl.BlockSpec((D, M), lambda i: (0, 0)),
        ],
        out_specs=pl.BlockSpec((_R, M), lambda i: (i, 0)),
        out_shape=jax.ShapeDtypeStruct((N, M), jnp.float32),
    )(agg, g, dinv, b.reshape(1, D), W)


def _tc_final(agg, g, dinv, b):
    N, D = g.shape
    H = D // 2

    def body(a_ref, g_ref, dinv_ref, b_ref, mu_ref, ls_ref):
        o = dinv_ref[...] * (a_ref[0] + a_ref[1] + g_ref[...]) + b_ref[...]
        mu_ref[...] = o[:, :H]
        ls_ref[...] = o[:, H:]

    return pl.pallas_call(
        body,
        grid=(N // _R,),
        in_specs=[
            pl.BlockSpec((NC, _R, D), lambda i: (0, i, 0)),
            pl.BlockSpec((_R, D), lambda i: (i, 0)),
            pl.BlockSpec((_R, 1), lambda i: (i, 0)),
            pl.BlockSpec((1, D), lambda i: (0, 0)),
        ],
        out_specs=[
            pl.BlockSpec((_R, H), lambda i: (i, 0)),
            pl.BlockSpec((_R, H), lambda i: (i, 0)),
        ],
        out_shape=[
            jax.ShapeDtypeStruct((N, H), jnp.float32),
            jax.ShapeDtypeStruct((N, H), jnp.float32),
        ],
    )(agg, g, dinv, b.reshape(1, D))


def kernel(x, edge_index, W1, b1, W2, b2, W_mu, b_mu, W_ls, b_ls):
    N, _ = x.shape
    E = edge_index.shape[1]
    NP = -(-N // (8 * NS)) * (8 * NS)
    EPW0 = E // NW
    EPW = -(-EPW0 // B) * B
    if (EPW // B) % 2 == 0:
        EPW += B

    src = jnp.pad(edge_index[0].reshape(NW, EPW0),
                  ((0, 0), (0, EPW - EPW0))).reshape(-1)
    npad = EPW - EPW0
    if npad:
        w = jnp.arange(NW)[:, None]
        i = jnp.arange(npad)[None, :]
        trash = N + (w * 37 + i) % (NP - N)
        dst = jnp.concatenate(
            [edge_index[1].reshape(NW, EPW0), trash.astype(jnp.int32)], axis=1
        ).reshape(NW, EPW // B, B)
    else:
        dst = edge_index[1].reshape(NW, EPW // B, B)
    zeros128 = jnp.zeros((NP, 128), jnp.float32)
    ones128 = jnp.ones((B, 128), jnp.float32)

    h1 = _tc_matmul(x, W1)
    degp = _make_deg(NP, EPW)(dst, zeros128, ones128)
    g1, dinv = _tc_scale(h1, degp)

    agg = _make_agg(NP, 128, EPW)
    a1 = agg(g1, src, dst, zeros128)
    g2 = _tc_mid(a1, g1, dinv, b1, W2)
    a2 = agg(g2, src, dst, zeros128)
    W3 = jnp.concatenate([W_mu, W_ls], axis=1)
    b3 = jnp.concatenate([b_mu, b_ls])
    g3 = _tc_mid(a2, g2, dinv, b2, W3)
    a3 = agg(g3, src, dst, zeros128)
    return _tc_final(a3, g3, dinv, b3)
